# Initial kernel scaffold; baseline (speedup 1.0000x reference)
import functools

import jax
import jax.numpy as jnp
from jax import lax
from jax.experimental import pallas as pl
from jax.experimental.pallas import tpu as pltpu
from jax.experimental.pallas import tpu_sc as plsc

NC = 2
NS = 16
NW = NC * NS
LANES = 16
CHUNK = 80


def _sc_mesh():
    return plsc.VectorSubcoreMesh(
        core_axis_name="c", subcore_axis_name="s", num_cores=NC, num_subcores=NS
    )


@functools.lru_cache(maxsize=None)
def _deg_call(n, e):
    epw = e // NW
    nchunks = epw // CHUNK
    npsc = n // NS

    @functools.partial(
        pl.kernel,
        out_type=jax.ShapeDtypeStruct((NC, n, LANES), jnp.float32),
        mesh=_sc_mesh(),
        scratch_types=[
            pltpu.VMEM((CHUNK,), jnp.int32),
            pltpu.VMEM((CHUNK,), jnp.float32),
            pltpu.VMEM((CHUNK, LANES), jnp.float32),
            pltpu.VMEM_SHARED((n, LANES), jnp.float32),
        ],
    )
    def k(dst_hbm, ew_hbm, zeros_hbm, out_hbm, dstv, ewv, rows, acc):
        cid = lax.axis_index("c")
        sid = lax.axis_index("s")
        wid = cid * NS + sid
        pltpu.sync_copy(zeros_hbm, acc.at[pl.ds(sid * npsc, npsc)])
        plsc.subcore_barrier()

        @pl.loop(0, nchunks)
        def _(ci):
            base = pl.multiple_of(wid * epw + ci * CHUNK, 8)
            pltpu.sync_copy(dst_hbm.at[pl.ds(base, CHUNK)], dstv)
            pltpu.sync_copy(ew_hbm.at[pl.ds(base, CHUNK)], ewv)

            @pl.loop(0, CHUNK)
            def _(r):
                rows[r, :] = jnp.broadcast_to(ewv[r], (LANES,))

            pltpu.sync_copy(rows, acc.at[dstv], add=True)

        plsc.subcore_barrier()
        pltpu.sync_copy(
            acc.at[pl.ds(sid * npsc, npsc)],
            out_hbm.at[cid, pl.ds(sid * npsc, npsc)],
        )

    return k


@functools.lru_cache(maxsize=None)
def _edge_call(n, e, f):
    epw = e // NW
    nchunks = epw // CHUNK
    npsc = n // NS

    @functools.partial(
        pl.kernel,
        out_type=jax.ShapeDtypeStruct((NC, n, f), jnp.float32),
        mesh=_sc_mesh(),
        scratch_types=[
            pltpu.VMEM((CHUNK,), jnp.int32),
            pltpu.VMEM((CHUNK,), jnp.int32),
            pltpu.VMEM((CHUNK,), jnp.float32),
            pltpu.VMEM((CHUNK, f), jnp.float32),
            pltpu.VMEM_SHARED((n, f), jnp.float32),
            pltpu.SemaphoreType.DMA,
        ],
    )
    def k(src_hbm, dst_hbm, ew_hbm, z_hbm, zeros_hbm, out_hbm,
          srcv, dstv, ewv, rows, acc, sem):
        cid = lax.axis_index("c")
        sid = lax.axis_index("s")
        wid = cid * NS + sid
        pltpu.sync_copy(zeros_hbm, acc.at[pl.ds(sid * npsc, npsc)])
        plsc.subcore_barrier()

        @pl.loop(0, nchunks)
        def _(ci):
            base = pl.multiple_of(wid * epw + ci * CHUNK, 8)
            pltpu.sync_copy(src_hbm.at[pl.ds(base, CHUNK)], srcv)
            pltpu.sync_copy(dst_hbm.at[pl.ds(base, CHUNK)], dstv)
            pltpu.sync_copy(ew_hbm.at[pl.ds(base, CHUNK)], ewv)
            pltpu.async_copy(z_hbm.at[srcv], rows, sem).wait()

            @pl.loop(0, CHUNK)
            def _(r):
                s = ewv[r]
                for j in range(f // LANES):
                    sl = pl.ds(j * LANES, LANES)
                    rows[r, sl] = rows[r, sl] * s

            pltpu.sync_copy(rows, acc.at[dstv], add=True)

        plsc.subcore_barrier()
        pltpu.sync_copy(
            acc.at[pl.ds(sid * npsc, npsc)],
            out_hbm.at[cid, pl.ds(sid * npsc, npsc)],
        )

    return k


def _prep_body(x_ref, w0_ref, wih_ref, bih_ref, bhh_ref, deg_ref, z_ref, w_scr):
    i = pl.program_id(0)

    @pl.when(i == 0)
    def _():
        gates = (
            jnp.dot(w0_ref[...], wih_ref[...].T, preferred_element_type=jnp.float32)
            + bih_ref[...]
            + bhh_ref[...]
        )
        fi = gates[:, 0:128]
        fg = gates[:, 256:384]
        fo = gates[:, 384:512]
        c = jax.nn.sigmoid(fi) * jnp.tanh(fg)
        w_scr[...] = jax.nn.sigmoid(fo) * jnp.tanh(c)

    xw = jnp.dot(x_ref[...], w_scr[...], preferred_element_type=jnp.float32)
    deg = deg_ref[0, :, 0] + deg_ref[1, :, 0] + 1.0
    dinv = lax.rsqrt(deg)
    z_ref[...] = xw * dinv[:, None]


def _prep(x, w0, wih, bih, bhh, deg16):
    n, f = x.shape
    blk = 1000
    grid = n // blk
    return pl.pallas_call(
        _prep_body,
        grid=(grid,),
        in_specs=[
            pl.BlockSpec((blk, f), lambda i: (i, 0)),
            pl.BlockSpec((f, f), lambda i: (0, 0)),
            pl.BlockSpec((4 * f, f), lambda i: (0, 0)),
            pl.BlockSpec((1, 4 * f), lambda i: (0, 0)),
            pl.BlockSpec((1, 4 * f), lambda i: (0, 0)),
            pl.BlockSpec((NC, blk, LANES), lambda i: (0, i, 0)),
        ],
        out_specs=pl.BlockSpec((blk, f), lambda i: (i, 0)),
        out_shape=jax.ShapeDtypeStruct((n, f), jnp.float32),
        scratch_shapes=[pltpu.VMEM((f, f), jnp.float32)],
    )(x, w0, wih, bih.reshape(1, 4 * f), bhh.reshape(1, 4 * f), deg16)


def _final_body(p_ref, z_ref, deg_ref, linw_ref, linb_ref, out_ref):
    deg = deg_ref[0, :, 0] + deg_ref[1, :, 0] + 1.0
    dinv = lax.rsqrt(deg)
    h = jnp.maximum((p_ref[0] + p_ref[1] + z_ref[...]) * dinv[:, None], 0.0)
    logits = (
        jnp.dot(h, linw_ref[...].T, preferred_element_type=jnp.float32)
        + linb_ref[...]
    )
    m = jnp.max(logits, axis=1, keepdims=True)
    ex = jnp.exp(logits - m)
    out_ref[...] = ex / jnp.sum(ex, axis=1, keepdims=True)


def _final(p, z, deg16, linw, linb):
    n, f = z.shape
    ncls = linw.shape[0]
    blk = 1000
    grid = n // blk
    return pl.pallas_call(
        _final_body,
        grid=(grid,),
        in_specs=[
            pl.BlockSpec((NC, blk, f), lambda i: (0, i, 0)),
            pl.BlockSpec((blk, f), lambda i: (i, 0)),
            pl.BlockSpec((NC, blk, LANES), lambda i: (0, i, 0)),
            pl.BlockSpec((ncls, f), lambda i: (0, 0)),
            pl.BlockSpec((1, ncls), lambda i: (0, 0)),
        ],
        out_specs=pl.BlockSpec((blk, ncls), lambda i: (i, 0)),
        out_shape=jax.ShapeDtypeStruct((n, ncls), jnp.float32),
    )(p, z, deg16, linw, linb.reshape(1, ncls))


def kernel(x, edge_index, edge_weight, initial_weight, W_ih, W_hh, b_ih, b_hh,
           lin_W, lin_b):
    n, f = x.shape
    e = edge_weight.shape[0]
    src = edge_index[0]
    dst = edge_index[1]
    zeros_deg = jnp.zeros((n // NS, LANES), jnp.float32)
    zeros_main = jnp.zeros((n // NS, f), jnp.float32)

    deg16 = _deg_call(n, e)(dst, edge_weight, zeros_deg)
    z = _prep(x, initial_weight, W_ih, b_ih, b_hh, deg16)
    p = _edge_call(n, e, f)(src, dst, edge_weight, z, zeros_main)
    return _final(p, z, deg16, lin_W, lin_b)


# speedup vs baseline: 12.5805x; 12.5805x over previous
import functools

import jax
import jax.numpy as jnp
from jax import lax
from jax.experimental import pallas as pl
from jax.experimental.pallas import tpu as pltpu
from jax.experimental.pallas import tpu_sc as plsc

NC = 2
NS = 16
NW = NC * NS
LANES = 16
CHUNK = 80


def _sc_mesh():
    return plsc.VectorSubcoreMesh(
        core_axis_name="c", subcore_axis_name="s", num_cores=NC, num_subcores=NS
    )


@functools.lru_cache(maxsize=None)
def _deg_call(n, e):
    epw = e // NW
    nchunks = epw // CHUNK
    nd = 10
    npsc = n // nd

    @functools.partial(
        pl.kernel,
        out_type=jax.ShapeDtypeStruct((NC, n, LANES), jnp.float32),
        mesh=_sc_mesh(),
        scratch_types=[
            pltpu.VMEM((CHUNK,), jnp.int32),
            pltpu.VMEM((CHUNK,), jnp.float32),
            pltpu.VMEM((CHUNK, LANES), jnp.float32),
            pltpu.VMEM_SHARED((n, LANES), jnp.float32),
        ],
    )
    def k(dst_hbm, ew_hbm, zeros_hbm, out_hbm, dstv, ewv, rows, acc):
        cid = lax.axis_index("c")
        sid = lax.axis_index("s")
        wid = cid * NS + sid

        @pl.when(sid < nd)
        def _():
            pltpu.sync_copy(zeros_hbm, acc.at[pl.ds(sid * npsc, npsc)])

        plsc.subcore_barrier()

        @pl.loop(0, nchunks)
        def _(ci):
            base = pl.multiple_of(wid * epw + ci * CHUNK, 8)
            pltpu.sync_copy(dst_hbm.at[pl.ds(base, CHUNK)], dstv)
            pltpu.sync_copy(ew_hbm.at[pl.ds(base, CHUNK)], ewv)

            @pl.loop(0, CHUNK, step=LANES)
            def _(r0):
                ev = ewv[pl.ds(r0, LANES)]
                for j in range(LANES):
                    rows[r0 + j, :] = jnp.broadcast_to(ev[j], (LANES,))

            pltpu.sync_copy(rows, acc.at[dstv], add=True)

        plsc.subcore_barrier()

        @pl.when(sid < nd)
        def _():
            pltpu.sync_copy(
                acc.at[pl.ds(sid * npsc, npsc)],
                out_hbm.at[cid, pl.ds(sid * npsc, npsc)],
            )

    return k


@functools.lru_cache(maxsize=None)
def _edge_call(n, e, f):
    epw = e // NW
    nchunks = epw // CHUNK
    nd = 10
    npsc = n // nd

    @functools.partial(
        pl.kernel,
        out_type=jax.ShapeDtypeStruct((NC, n, f), jnp.float32),
        mesh=_sc_mesh(),
        scratch_types=[
            pltpu.VMEM((CHUNK,), jnp.int32),
            pltpu.VMEM((CHUNK,), jnp.int32),
            pltpu.VMEM((CHUNK,), jnp.float32),
            pltpu.VMEM((CHUNK, f), jnp.float32),
            pltpu.VMEM_SHARED((n, f), jnp.float32),
            pltpu.SemaphoreType.DMA,
        ],
    )
    def k(src_hbm, dst_hbm, ew_hbm, z_hbm, zeros_hbm, out_hbm,
          srcv, dstv, ewv, rows, acc, sem):
        cid = lax.axis_index("c")
        sid = lax.axis_index("s")
        wid = cid * NS + sid

        @pl.when(sid < nd)
        def _():
            pltpu.sync_copy(zeros_hbm, acc.at[pl.ds(sid * npsc, npsc)])

        plsc.subcore_barrier()

        @pl.loop(0, nchunks)
        def _(ci):
            base = pl.multiple_of(wid * epw + ci * CHUNK, 8)
            pltpu.sync_copy(src_hbm.at[pl.ds(base, CHUNK)], srcv)
            pltpu.sync_copy(dst_hbm.at[pl.ds(base, CHUNK)], dstv)
            pltpu.sync_copy(ew_hbm.at[pl.ds(base, CHUNK)], ewv)
            pltpu.async_copy(z_hbm.at[srcv], rows, sem).wait()

            @pl.loop(0, CHUNK, step=LANES)
            def _(r0):
                ev = ewv[pl.ds(r0, LANES)]
                for j in range(LANES):
                    s = ev[j]
                    for k in range(f // LANES):
                        sl = pl.ds(k * LANES, LANES)
                        rows[r0 + j, sl] = rows[r0 + j, sl] * s

            pltpu.sync_copy(rows, acc.at[dstv], add=True)

        plsc.subcore_barrier()

        @pl.when(sid < nd)
        def _():
            pltpu.sync_copy(
                acc.at[pl.ds(sid * npsc, npsc)],
                out_hbm.at[cid, pl.ds(sid * npsc, npsc)],
            )

    return k


def _prep_body(x_ref, w0_ref, wih_ref, bih_ref, bhh_ref, deg_ref, z_ref, w_scr):
    i = pl.program_id(0)

    @pl.when(i == 0)
    def _():
        gates = (
            jnp.dot(w0_ref[...], wih_ref[...].T, preferred_element_type=jnp.float32)
            + bih_ref[...]
            + bhh_ref[...]
        )
        fi = gates[:, 0:128]
        fg = gates[:, 256:384]
        fo = gates[:, 384:512]
        c = jax.nn.sigmoid(fi) * jnp.tanh(fg)
        w_scr[...] = jax.nn.sigmoid(fo) * jnp.tanh(c)

    xw = jnp.dot(x_ref[...], w_scr[...], preferred_element_type=jnp.float32)
    deg = deg_ref[0, :, 0] + deg_ref[1, :, 0] + 1.0
    dinv = lax.rsqrt(deg)
    z_ref[...] = xw * dinv[:, None]


def _prep(x, w0, wih, bih, bhh, deg16):
    n, f = x.shape
    blk = 1000
    grid = n // blk
    return pl.pallas_call(
        _prep_body,
        grid=(grid,),
        in_specs=[
            pl.BlockSpec((blk, f), lambda i: (i, 0)),
            pl.BlockSpec((f, f), lambda i: (0, 0)),
            pl.BlockSpec((4 * f, f), lambda i: (0, 0)),
            pl.BlockSpec((1, 4 * f), lambda i: (0, 0)),
            pl.BlockSpec((1, 4 * f), lambda i: (0, 0)),
            pl.BlockSpec((NC, blk, LANES), lambda i: (0, i, 0)),
        ],
        out_specs=pl.BlockSpec((blk, f), lambda i: (i, 0)),
        out_shape=jax.ShapeDtypeStruct((n, f), jnp.float32),
        scratch_shapes=[pltpu.VMEM((f, f), jnp.float32)],
    )(x, w0, wih, bih.reshape(1, 4 * f), bhh.reshape(1, 4 * f), deg16)


def _final_body(p_ref, z_ref, deg_ref, linw_ref, linb_ref, out_ref):
    deg = deg_ref[0, :, 0] + deg_ref[1, :, 0] + 1.0
    dinv = lax.rsqrt(deg)
    h = jnp.maximum((p_ref[0] + p_ref[1] + z_ref[...]) * dinv[:, None], 0.0)
    logits = (
        jnp.dot(h, linw_ref[...].T, preferred_element_type=jnp.float32)
        + linb_ref[...]
    )
    m = jnp.max(logits, axis=1, keepdims=True)
    ex = jnp.exp(logits - m)
    out_ref[...] = ex / jnp.sum(ex, axis=1, keepdims=True)


def _final(p, z, deg16, linw, linb):
    n, f = z.shape
    ncls = linw.shape[0]
    blk = 1000
    grid = n // blk
    return pl.pallas_call(
        _final_body,
        grid=(grid,),
        in_specs=[
            pl.BlockSpec((NC, blk, f), lambda i: (0, i, 0)),
            pl.BlockSpec((blk, f), lambda i: (i, 0)),
            pl.BlockSpec((NC, blk, LANES), lambda i: (0, i, 0)),
            pl.BlockSpec((ncls, f), lambda i: (0, 0)),
            pl.BlockSpec((1, ncls), lambda i: (0, 0)),
        ],
        out_specs=pl.BlockSpec((blk, ncls), lambda i: (i, 0)),
        out_shape=jax.ShapeDtypeStruct((n, ncls), jnp.float32),
    )(p, z, deg16, linw, linb.reshape(1, ncls))


def kernel(x, edge_index, edge_weight, initial_weight, W_ih, W_hh, b_ih, b_hh,
           lin_W, lin_b):
    n, f = x.shape
    e = edge_weight.shape[0]
    src = edge_index[0]
    dst = edge_index[1]
    zeros_deg = jnp.zeros((n // 10, LANES), jnp.float32)
    zeros_main = jnp.zeros((n // 10, f), jnp.float32)

    deg16 = _deg_call(n, e)(dst, edge_weight, zeros_deg)
    z = _prep(x, initial_weight, W_ih, b_ih, b_hh, deg16)
    p = _edge_call(n, e, f)(src, dst, edge_weight, z, zeros_main)
    return _final(p, z, deg16, lin_W, lin_b)


# speedup vs baseline: 22.3566x; 1.7771x over previous
import functools

import jax
import jax.numpy as jnp
from jax import lax
from jax.experimental import pallas as pl
from jax.experimental.pallas import tpu as pltpu
from jax.experimental.pallas import tpu_sc as plsc

NC = 2
NS = 16
NW = NC * NS
LANES = 16
CHUNK = 80
ND = 10


def _sc_mesh():
    return plsc.VectorSubcoreMesh(
        core_axis_name="c", subcore_axis_name="s", num_cores=NC, num_subcores=NS
    )


@functools.lru_cache(maxsize=None)
def _deg_call(n, e):
    epw = e // NW
    nchunks = epw // CHUNK
    assert nchunks % 2 == 1
    npairs = nchunks // 2
    npsc = n // ND

    @functools.partial(
        pl.kernel,
        out_type=jax.ShapeDtypeStruct((NC, n, LANES), jnp.float32),
        mesh=_sc_mesh(),
        scratch_types=[
            pltpu.VMEM((epw,), jnp.int32),
            pltpu.VMEM((epw,), jnp.float32),
            pltpu.VMEM((CHUNK, LANES), jnp.float32),
            pltpu.VMEM((CHUNK, LANES), jnp.float32),
            pltpu.VMEM((CHUNK,), jnp.int32),
            pltpu.VMEM((CHUNK,), jnp.int32),
            pltpu.VMEM_SHARED((n, LANES), jnp.float32),
            pltpu.SemaphoreType.DMA,
            pltpu.SemaphoreType.DMA,
            pltpu.SemaphoreType.DMA,
        ],
    )
    def k(dst_hbm, ew_hbm, zeros_hbm, out_hbm,
          dsts, ews, r0buf, r1buf, d0, d1, acc, stsem, ssem0, ssem1):
        cid = lax.axis_index("c")
        sid = lax.axis_index("s")
        wid = cid * NS + sid
        base = pl.multiple_of(wid * epw, 8)

        pltpu.async_copy(dst_hbm.at[pl.ds(base, epw)], dsts, stsem)
        pltpu.async_copy(ew_hbm.at[pl.ds(base, epw)], ews, stsem)

        @pl.when(sid < ND)
        def _():
            pltpu.async_copy(zeros_hbm, acc.at[pl.ds(sid * npsc, npsc)], stsem)

        pltpu.make_async_copy(dst_hbm.at[pl.ds(base, epw)], dsts, stsem).wait()
        pltpu.make_async_copy(ew_hbm.at[pl.ds(base, epw)], ews, stsem).wait()

        @pl.when(sid < ND)
        def _():
            pltpu.make_async_copy(
                zeros_hbm, acc.at[pl.ds(sid * npsc, npsc)], stsem
            ).wait()

        plsc.subcore_barrier()

        def build(ci, rbuf, dcur):
            @pl.loop(0, CHUNK, step=LANES)
            def _(r0):
                ev = ews[pl.ds(ci * CHUNK + r0, LANES)]
                for j in range(LANES):
                    rbuf[r0 + j, :] = jnp.broadcast_to(ev[j], (LANES,))

            for t in range(CHUNK // LANES):
                sl = pl.ds(t * LANES, LANES)
                dcur[sl] = dsts[pl.ds(ci * CHUNK + t * LANES, LANES)]

        def slot(ci, rbuf, dcur, ssem, first):
            if not first:
                pltpu.make_async_copy(rbuf, acc.at[dcur], ssem).wait()
            build(ci, rbuf, dcur)
            pltpu.async_copy(rbuf, acc.at[dcur], ssem, add=True)

        slot(0, r0buf, d0, ssem0, True)
        slot(1, r1buf, d1, ssem1, True)

        @pl.loop(1, npairs)
        def _(g):
            slot(2 * g, r0buf, d0, ssem0, False)
            slot(2 * g + 1, r1buf, d1, ssem1, False)

        pltpu.make_async_copy(r0buf, acc.at[d0], ssem0).wait()
        build(nchunks - 1, r0buf, d0)
        pltpu.sync_copy(r0buf, acc.at[d0], add=True)
        pltpu.make_async_copy(r1buf, acc.at[d1], ssem1).wait()

        plsc.subcore_barrier()

        @pl.when(sid < ND)
        def _():
            pltpu.sync_copy(
                acc.at[pl.ds(sid * npsc, npsc)],
                out_hbm.at[cid, pl.ds(sid * npsc, npsc)],
            )

    return k


@functools.lru_cache(maxsize=None)
def _edge_call(n, e, f):
    epw = e // NW
    nchunks = epw // CHUNK
    assert nchunks % 2 == 1
    npairs = nchunks // 2
    npsc = n // ND

    @functools.partial(
        pl.kernel,
        out_type=jax.ShapeDtypeStruct((NC, n, f), jnp.float32),
        mesh=_sc_mesh(),
        scratch_types=[
            pltpu.VMEM((epw,), jnp.int32),
            pltpu.VMEM((epw,), jnp.int32),
            pltpu.VMEM((epw,), jnp.float32),
            pltpu.VMEM((CHUNK, f), jnp.float32),
            pltpu.VMEM((CHUNK,), jnp.int32),
            pltpu.VMEM_SHARED((n, f), jnp.float32),
            pltpu.SemaphoreType.DMA,
            pltpu.SemaphoreType.DMA,
        ],
    )
    def k(src_hbm, dst_hbm, ew_hbm, z_hbm, zeros_hbm, out_hbm,
          srcs, dsts, ews, rows, d0, acc, stsem, gsem):
        cid = lax.axis_index("c")
        sid = lax.axis_index("s")
        wid = cid * NS + sid
        base = pl.multiple_of(wid * epw, 8)

        pltpu.async_copy(src_hbm.at[pl.ds(base, epw)], srcs, stsem)
        pltpu.async_copy(dst_hbm.at[pl.ds(base, epw)], dsts, stsem)
        pltpu.async_copy(ew_hbm.at[pl.ds(base, epw)], ews, stsem)

        @pl.when(sid < ND)
        def _():
            pltpu.async_copy(zeros_hbm, acc.at[pl.ds(sid * npsc, npsc)], stsem)

        pltpu.make_async_copy(src_hbm.at[pl.ds(base, epw)], srcs, stsem).wait()
        pltpu.make_async_copy(dst_hbm.at[pl.ds(base, epw)], dsts, stsem).wait()
        pltpu.make_async_copy(ew_hbm.at[pl.ds(base, epw)], ews, stsem).wait()

        @pl.when(sid < ND)
        def _():
            pltpu.make_async_copy(
                zeros_hbm, acc.at[pl.ds(sid * npsc, npsc)], stsem
            ).wait()

        plsc.subcore_barrier()

        @pl.loop(0, nchunks)
        def _(ci):
            pltpu.async_copy(
                z_hbm.at[srcs.at[pl.ds(ci * CHUNK, CHUNK)]], rows, gsem
            ).wait()

            @pl.loop(0, CHUNK, step=LANES)
            def _(r0):
                ev = ews[pl.ds(ci * CHUNK + r0, LANES)]
                for j in range(LANES):
                    s = ev[j]
                    for kk in range(f // LANES):
                        sl = pl.ds(kk * LANES, LANES)
                        rows[r0 + j, sl] = rows[r0 + j, sl] * s

            for t in range(CHUNK // LANES):
                sl = pl.ds(t * LANES, LANES)
                d0[sl] = dsts[pl.ds(ci * CHUNK + t * LANES, LANES)]

            pltpu.sync_copy(rows, acc.at[d0], add=True)

        plsc.subcore_barrier()

        @pl.when(sid < ND)
        def _():
            pltpu.sync_copy(
                acc.at[pl.ds(sid * npsc, npsc)],
                out_hbm.at[cid, pl.ds(sid * npsc, npsc)],
            )

    return k


def _prep_body(x_ref, w0_ref, wih_ref, bih_ref, bhh_ref, deg_ref, z_ref, w_scr):
    i = pl.program_id(0)

    @pl.when(i == 0)
    def _():
        gates = (
            jnp.dot(w0_ref[...], wih_ref[...].T, preferred_element_type=jnp.float32)
            + bih_ref[...]
            + bhh_ref[...]
        )
        fi = gates[:, 0:128]
        fg = gates[:, 256:384]
        fo = gates[:, 384:512]
        c = jax.nn.sigmoid(fi) * jnp.tanh(fg)
        w_scr[...] = jax.nn.sigmoid(fo) * jnp.tanh(c)

    xw = jnp.dot(x_ref[...], w_scr[...], preferred_element_type=jnp.float32)
    deg = deg_ref[0, :, 0] + deg_ref[1, :, 0] + 1.0
    dinv = lax.rsqrt(deg)
    z_ref[...] = xw * dinv[:, None]


def _prep(x, w0, wih, bih, bhh, deg16):
    n, f = x.shape
    blk = 1000
    grid = n // blk
    return pl.pallas_call(
        _prep_body,
        grid=(grid,),
        in_specs=[
            pl.BlockSpec((blk, f), lambda i: (i, 0)),
            pl.BlockSpec((f, f), lambda i: (0, 0)),
            pl.BlockSpec((4 * f, f), lambda i: (0, 0)),
            pl.BlockSpec((1, 4 * f), lambda i: (0, 0)),
            pl.BlockSpec((1, 4 * f), lambda i: (0, 0)),
            pl.BlockSpec((NC, blk, LANES), lambda i: (0, i, 0)),
        ],
        out_specs=pl.BlockSpec((blk, f), lambda i: (i, 0)),
        out_shape=jax.ShapeDtypeStruct((n, f), jnp.float32),
        scratch_shapes=[pltpu.VMEM((f, f), jnp.float32)],
    )(x, w0, wih, bih.reshape(1, 4 * f), bhh.reshape(1, 4 * f), deg16)


def _final_body(p_ref, z_ref, deg_ref, linw_ref, linb_ref, out_ref):
    deg = deg_ref[0, :, 0] + deg_ref[1, :, 0] + 1.0
    dinv = lax.rsqrt(deg)
    h = jnp.maximum((p_ref[0] + p_ref[1] + z_ref[...]) * dinv[:, None], 0.0)
    logits = (
        jnp.dot(h, linw_ref[...].T, preferred_element_type=jnp.float32)
        + linb_ref[...]
    )
    m = jnp.max(logits, axis=1, keepdims=True)
    ex = jnp.exp(logits - m)
    out_ref[...] = ex / jnp.sum(ex, axis=1, keepdims=True)


def _final(p, z, deg16, linw, linb):
    n, f = z.shape
    ncls = linw.shape[0]
    blk = 1000
    grid = n // blk
    return pl.pallas_call(
        _final_body,
        grid=(grid,),
        in_specs=[
            pl.BlockSpec((NC, blk, f), lambda i: (0, i, 0)),
            pl.BlockSpec((blk, f), lambda i: (i, 0)),
            pl.BlockSpec((NC, blk, LANES), lambda i: (0, i, 0)),
            pl.BlockSpec((ncls, f), lambda i: (0, 0)),
            pl.BlockSpec((1, ncls), lambda i: (0, 0)),
        ],
        out_specs=pl.BlockSpec((blk, ncls), lambda i: (i, 0)),
        out_shape=jax.ShapeDtypeStruct((n, ncls), jnp.float32),
    )(p, z, deg16, linw, linb.reshape(1, ncls))


def kernel(x, edge_index, edge_weight, initial_weight, W_ih, W_hh, b_ih, b_hh,
           lin_W, lin_b):
    n, f = x.shape
    e = edge_weight.shape[0]
    src = edge_index[0]
    dst = edge_index[1]
    zeros_deg = jnp.zeros((n // ND, LANES), jnp.float32)
    zeros_main = jnp.zeros((n // ND, f), jnp.float32)

    deg16 = _deg_call(n, e)(dst, edge_weight, zeros_deg)
    z = _prep(x, initial_weight, W_ih, b_ih, b_hh, deg16)
    p = _edge_call(n, e, f)(src, dst, edge_weight, z, zeros_main)
    return _final(p, z, deg16, lin_W, lin_b)


# speedup vs baseline: 31.9274x; 1.4281x over previous
import functools

import jax
import jax.numpy as jnp
from jax import lax
from jax.experimental import pallas as pl
from jax.experimental.pallas import tpu as pltpu
from jax.experimental.pallas import tpu_sc as plsc

NC = 2
NS = 16
NW = NC * NS
LANES = 16
CHUNK = 80
ND = 10


def _sc_mesh():
    return plsc.VectorSubcoreMesh(
        core_axis_name="c", subcore_axis_name="s", num_cores=NC, num_subcores=NS
    )


@functools.lru_cache(maxsize=None)
def _deg_call(n, e):
    epw = e // NW
    nchunks = epw // CHUNK
    assert nchunks % 2 == 1
    npairs = nchunks // 2
    npsc = n // ND

    @functools.partial(
        pl.kernel,
        out_type=jax.ShapeDtypeStruct((NC, n, LANES), jnp.float32),
        mesh=_sc_mesh(),
        scratch_types=[
            pltpu.VMEM((epw,), jnp.int32),
            pltpu.VMEM((epw,), jnp.float32),
            pltpu.VMEM((CHUNK, LANES), jnp.float32),
            pltpu.VMEM((CHUNK, LANES), jnp.float32),
            pltpu.VMEM((CHUNK,), jnp.int32),
            pltpu.VMEM((CHUNK,), jnp.int32),
            pltpu.VMEM_SHARED((n, LANES), jnp.float32),
            pltpu.SemaphoreType.DMA,
            pltpu.SemaphoreType.DMA,
            pltpu.SemaphoreType.DMA,
        ],
    )
    def k(dst_hbm, ew_hbm, zeros_hbm, out_hbm,
          dsts, ews, r0buf, r1buf, d0, d1, acc, stsem, ssem0, ssem1):
        cid = lax.axis_index("c")
        sid = lax.axis_index("s")
        wid = cid * NS + sid
        base = pl.multiple_of(wid * epw, 8)

        pltpu.async_copy(dst_hbm.at[pl.ds(base, epw)], dsts, stsem)
        pltpu.async_copy(ew_hbm.at[pl.ds(base, epw)], ews, stsem)

        @pl.when(sid < ND)
        def _():
            pltpu.async_copy(zeros_hbm, acc.at[pl.ds(sid * npsc, npsc)], stsem)

        pltpu.make_async_copy(dst_hbm.at[pl.ds(base, epw)], dsts, stsem).wait()
        pltpu.make_async_copy(ew_hbm.at[pl.ds(base, epw)], ews, stsem).wait()

        @pl.when(sid < ND)
        def _():
            pltpu.make_async_copy(
                zeros_hbm, acc.at[pl.ds(sid * npsc, npsc)], stsem
            ).wait()

        plsc.subcore_barrier()

        def build(ci, rbuf, dcur):
            @pl.loop(0, CHUNK, step=LANES)
            def _(r0):
                ev = ews[pl.ds(ci * CHUNK + r0, LANES)]
                for j in range(LANES):
                    rbuf[r0 + j, :] = jnp.broadcast_to(ev[j], (LANES,))

            for t in range(CHUNK // LANES):
                sl = pl.ds(t * LANES, LANES)
                dcur[sl] = dsts[pl.ds(ci * CHUNK + t * LANES, LANES)]

        def slot(ci, rbuf, dcur, ssem, first):
            if not first:
                pltpu.make_async_copy(rbuf, acc.at[dcur], ssem).wait()
            build(ci, rbuf, dcur)
            pltpu.async_copy(rbuf, acc.at[dcur], ssem, add=True)

        slot(0, r0buf, d0, ssem0, True)
        slot(1, r1buf, d1, ssem1, True)

        @pl.loop(1, npairs)
        def _(g):
            slot(2 * g, r0buf, d0, ssem0, False)
            slot(2 * g + 1, r1buf, d1, ssem1, False)

        pltpu.make_async_copy(r0buf, acc.at[d0], ssem0).wait()
        build(nchunks - 1, r0buf, d0)
        pltpu.sync_copy(r0buf, acc.at[d0], add=True)
        pltpu.make_async_copy(r1buf, acc.at[d1], ssem1).wait()

        plsc.subcore_barrier()

        @pl.when(sid < ND)
        def _():
            pltpu.sync_copy(
                acc.at[pl.ds(sid * npsc, npsc)],
                out_hbm.at[cid, pl.ds(sid * npsc, npsc)],
            )

    return k


NR = 4
NSLOT = 8


@functools.lru_cache(maxsize=None)
def _edge_call(n, e, f):
    epw = e // NW
    nchunks = epw // CHUNK
    npsc = n // ND
    assert nchunks >= 2 * NSLOT
    body_end = ((nchunks - 2) // NSLOT) * NSLOT

    @functools.partial(
        pl.kernel,
        out_type=jax.ShapeDtypeStruct((NC, n, f), jnp.float32),
        mesh=_sc_mesh(),
        scratch_types=(
            [pltpu.VMEM((CHUNK, f), jnp.float32)] * NR
            + [pltpu.VMEM((CHUNK,), jnp.int32)] * NSLOT
            + [pltpu.VMEM((CHUNK,), jnp.int32)] * NSLOT
            + [pltpu.VMEM((CHUNK,), jnp.float32)] * NSLOT
            + [
                pltpu.VMEM_SHARED((n, f), jnp.float32),
                pltpu.SemaphoreType.DMA,
            ]
            + [pltpu.SemaphoreType.DMA] * NR
            + [pltpu.SemaphoreType.DMA] * NR
            + [pltpu.SemaphoreType.DMA] * NSLOT
        ),
    )
    def k(src_hbm, dst_hbm, ew_hbm, z_hbm, zeros_hbm, out_hbm, *rest):
        rows = list(rest[0:NR])
        o = NR
        srcb = list(rest[o:o + NSLOT]); o += NSLOT
        dstb = list(rest[o:o + NSLOT]); o += NSLOT
        ewb = list(rest[o:o + NSLOT]); o += NSLOT
        acc = rest[o]; o += 1
        zsem = rest[o]; o += 1
        gsem = list(rest[o:o + NR]); o += NR
        ssem = list(rest[o:o + NR]); o += NR
        stsem = list(rest[o:o + NSLOT])

        cid = lax.axis_index("c")
        sid = lax.axis_index("s")
        wid = cid * NS + sid
        base = pl.multiple_of(wid * epw, 8)

        def stissue(ci, c):
            pltpu.async_copy(src_hbm.at[pl.ds(base + ci * CHUNK, CHUNK)],
                             srcb[c], stsem[c])
            pltpu.async_copy(dst_hbm.at[pl.ds(base + ci * CHUNK, CHUNK)],
                             dstb[c], stsem[c])
            pltpu.async_copy(ew_hbm.at[pl.ds(base + ci * CHUNK, CHUNK)],
                             ewb[c], stsem[c])

        def stwait(c):
            pltpu.make_async_copy(src_hbm.at[pl.ds(base, CHUNK)],
                                  srcb[c], stsem[c]).wait()
            pltpu.make_async_copy(dst_hbm.at[pl.ds(base, CHUNK)],
                                  dstb[c], stsem[c]).wait()
            pltpu.make_async_copy(ew_hbm.at[pl.ds(base, CHUNK)],
                                  ewb[c], stsem[c]).wait()

        def gissue(b, c):
            pltpu.async_copy(z_hbm.at[srcb[c]], rows[b], gsem[b])

        def gwait(b, c):
            pltpu.make_async_copy(z_hbm.at[srcb[c]], rows[b], gsem[b]).wait()

        def scale(b, c):
            @pl.loop(0, CHUNK, step=LANES)
            def _(r0):
                ev = ewb[c][pl.ds(r0, LANES)]
                for j in range(LANES):
                    s = ev[j]
                    for kk in range(f // LANES):
                        sl = pl.ds(kk * LANES, LANES)
                        rows[b][r0 + j, sl] = rows[b][r0 + j, sl] * s

        def sissue(b, c):
            pltpu.sync_copy(rows[b], acc.at[dstb[c]], add=True)

        def swait(b, c):
            del b, c

        def emit(ci, k_, tail):
            b = k_ % NR
            gwait(b, k_)
            scale(b, k_)
            sissue(b, k_)
            if not tail:
                @pl.when(ci >= 2)
                def _():
                    swait((k_ + 2) % NR, (k_ + 6) % NSLOT)

                stwait((k_ + 2) % NSLOT)
                gissue((k_ + 2) % NR, (k_ + 2) % NSLOT)

                @pl.when(ci + 6 < nchunks)
                def _():
                    stissue(ci + 6, (k_ + 6) % NSLOT)
            else:
                swait((k_ + 2) % NR, (k_ + 6) % NSLOT)
                if ci + 2 < nchunks:
                    stwait((k_ + 2) % NSLOT)
                    gissue((k_ + 2) % NR, (k_ + 2) % NSLOT)

        @pl.when(sid < ND)
        def _():
            pltpu.async_copy(zeros_hbm, acc.at[pl.ds(sid * npsc, npsc)], zsem)

        for cj in range(6):
            stissue(cj, cj)

        @pl.when(sid < ND)
        def _():
            pltpu.make_async_copy(
                zeros_hbm, acc.at[pl.ds(sid * npsc, npsc)], zsem
            ).wait()

        plsc.subcore_barrier()

        for cj in range(2):
            stwait(cj)
            gissue(cj, cj)

        @pl.loop(0, body_end // NSLOT)
        def _(g):
            for k_ in range(NSLOT):
                emit(g * NSLOT + k_, k_, False)

        for ci in range(body_end, nchunks):
            emit(ci, ci % NSLOT, True)

        for cj in (nchunks - 2, nchunks - 1):
            swait(cj % NR, cj % NSLOT)

        plsc.subcore_barrier()

        @pl.when(sid < ND)
        def _():
            pltpu.sync_copy(
                acc.at[pl.ds(sid * npsc, npsc)],
                out_hbm.at[cid, pl.ds(sid * npsc, npsc)],
            )

    return k


def _prep_body(x_ref, w0_ref, wih_ref, bih_ref, bhh_ref, deg_ref, z_ref, w_scr):
    i = pl.program_id(0)

    @pl.when(i == 0)
    def _():
        gates = (
            jnp.dot(w0_ref[...], wih_ref[...].T, preferred_element_type=jnp.float32)
            + bih_ref[...]
            + bhh_ref[...]
        )
        fi = gates[:, 0:128]
        fg = gates[:, 256:384]
        fo = gates[:, 384:512]
        c = jax.nn.sigmoid(fi) * jnp.tanh(fg)
        w_scr[...] = jax.nn.sigmoid(fo) * jnp.tanh(c)

    xw = jnp.dot(x_ref[...], w_scr[...], preferred_element_type=jnp.float32)
    deg = deg_ref[0, :, 0] + deg_ref[1, :, 0] + 1.0
    dinv = lax.rsqrt(deg)
    z_ref[...] = xw * dinv[:, None]


def _prep(x, w0, wih, bih, bhh, deg16):
    n, f = x.shape
    blk = 1000
    grid = n // blk
    return pl.pallas_call(
        _prep_body,
        grid=(grid,),
        in_specs=[
            pl.BlockSpec((blk, f), lambda i: (i, 0)),
            pl.BlockSpec((f, f), lambda i: (0, 0)),
            pl.BlockSpec((4 * f, f), lambda i: (0, 0)),
            pl.BlockSpec((1, 4 * f), lambda i: (0, 0)),
            pl.BlockSpec((1, 4 * f), lambda i: (0, 0)),
            pl.BlockSpec((NC, blk, LANES), lambda i: (0, i, 0)),
        ],
        out_specs=pl.BlockSpec((blk, f), lambda i: (i, 0)),
        out_shape=jax.ShapeDtypeStruct((n, f), jnp.float32),
        scratch_shapes=[pltpu.VMEM((f, f), jnp.float32)],
    )(x, w0, wih, bih.reshape(1, 4 * f), bhh.reshape(1, 4 * f), deg16)


def _final_body(p_ref, z_ref, deg_ref, linw_ref, linb_ref, out_ref):
    deg = deg_ref[0, :, 0] + deg_ref[1, :, 0] + 1.0
    dinv = lax.rsqrt(deg)
    h = jnp.maximum((p_ref[0] + p_ref[1] + z_ref[...]) * dinv[:, None], 0.0)
    logits = (
        jnp.dot(h, linw_ref[...].T, preferred_element_type=jnp.float32)
        + linb_ref[...]
    )
    m = jnp.max(logits, axis=1, keepdims=True)
    ex = jnp.exp(logits - m)
    out_ref[...] = ex / jnp.sum(ex, axis=1, keepdims=True)


def _final(p, z, deg16, linw, linb):
    n, f = z.shape
    ncls = linw.shape[0]
    blk = 1000
    grid = n // blk
    return pl.pallas_call(
        _final_body,
        grid=(grid,),
        in_specs=[
            pl.BlockSpec((NC, blk, f), lambda i: (0, i, 0)),
            pl.BlockSpec((blk, f), lambda i: (i, 0)),
            pl.BlockSpec((NC, blk, LANES), lambda i: (0, i, 0)),
            pl.BlockSpec((ncls, f), lambda i: (0, 0)),
            pl.BlockSpec((1, ncls), lambda i: (0, 0)),
        ],
        out_specs=pl.BlockSpec((blk, ncls), lambda i: (i, 0)),
        out_shape=jax.ShapeDtypeStruct((n, ncls), jnp.float32),
    )(p, z, deg16, linw, linb.reshape(1, ncls))


def kernel(x, edge_index, edge_weight, initial_weight, W_ih, W_hh, b_ih, b_hh,
           lin_W, lin_b):
    n, f = x.shape
    e = edge_weight.shape[0]
    src = edge_index[0]
    dst = edge_index[1]
    zeros_deg = jnp.zeros((n // ND, LANES), jnp.float32)
    zeros_main = jnp.zeros((n // ND, f), jnp.float32)

    deg16 = _deg_call(n, e)(dst, edge_weight, zeros_deg)
    z = _prep(x, initial_weight, W_ih, b_ih, b_hh, deg16)
    p = _edge_call(n, e, f)(src, dst, edge_weight, z, zeros_main)
    return _final(p, z, deg16, lin_W, lin_b)


# speedup vs baseline: 36.4618x; 1.1420x over previous
import functools

import jax
import jax.numpy as jnp
from jax import lax
from jax.experimental import pallas as pl
from jax.experimental.pallas import tpu as pltpu
from jax.experimental.pallas import tpu_sc as plsc

NC = 2
NS = 16
NW = NC * NS
LANES = 16
CHUNK = 80
ND = 10


def _sc_mesh():
    return plsc.VectorSubcoreMesh(
        core_axis_name="c", subcore_axis_name="s", num_cores=NC, num_subcores=NS
    )


@functools.lru_cache(maxsize=None)
def _deg_call(n, e):
    epw = e // NW
    nchunks = epw // CHUNK
    assert nchunks % 2 == 1
    npairs = nchunks // 2
    npsc = n // ND

    @functools.partial(
        pl.kernel,
        out_type=jax.ShapeDtypeStruct((NC, n, LANES), jnp.float32),
        mesh=_sc_mesh(),
        scratch_types=[
            pltpu.VMEM((epw,), jnp.int32),
            pltpu.VMEM((epw,), jnp.float32),
            pltpu.VMEM((CHUNK, LANES), jnp.float32),
            pltpu.VMEM((CHUNK, LANES), jnp.float32),
            pltpu.VMEM((CHUNK,), jnp.int32),
            pltpu.VMEM((CHUNK,), jnp.int32),
            pltpu.VMEM_SHARED((n, LANES), jnp.float32),
            pltpu.SemaphoreType.DMA,
            pltpu.SemaphoreType.DMA,
            pltpu.SemaphoreType.DMA,
        ],
    )
    def k(dst_hbm, ew_hbm, zeros_hbm, out_hbm,
          dsts, ews, r0buf, r1buf, d0, d1, acc, stsem, ssem0, ssem1):
        cid = lax.axis_index("c")
        sid = lax.axis_index("s")
        wid = cid * NS + sid
        base = pl.multiple_of(wid * epw, 8)

        pltpu.async_copy(dst_hbm.at[pl.ds(base, epw)], dsts, stsem)
        pltpu.async_copy(ew_hbm.at[pl.ds(base, epw)], ews, stsem)

        @pl.when(sid < ND)
        def _():
            pltpu.async_copy(zeros_hbm, acc.at[pl.ds(sid * npsc, npsc)], stsem)

        pltpu.make_async_copy(dst_hbm.at[pl.ds(base, epw)], dsts, stsem).wait()
        pltpu.make_async_copy(ew_hbm.at[pl.ds(base, epw)], ews, stsem).wait()

        @pl.when(sid < ND)
        def _():
            pltpu.make_async_copy(
                zeros_hbm, acc.at[pl.ds(sid * npsc, npsc)], stsem
            ).wait()

        plsc.subcore_barrier()

        def build(ci, rbuf, dcur):
            @pl.loop(0, CHUNK, step=LANES)
            def _(r0):
                ev = ews[pl.ds(ci * CHUNK + r0, LANES)]
                for j in range(LANES):
                    rbuf[r0 + j, :] = jnp.broadcast_to(ev[j], (LANES,))

            for t in range(CHUNK // LANES):
                sl = pl.ds(t * LANES, LANES)
                dcur[sl] = dsts[pl.ds(ci * CHUNK + t * LANES, LANES)]

        def slot(ci, rbuf, dcur, ssem, first):
            if not first:
                pltpu.make_async_copy(rbuf, acc.at[dcur], ssem).wait()
            build(ci, rbuf, dcur)
            pltpu.async_copy(rbuf, acc.at[dcur], ssem, add=True)

        slot(0, r0buf, d0, ssem0, True)
        slot(1, r1buf, d1, ssem1, True)

        @pl.loop(1, npairs)
        def _(g):
            slot(2 * g, r0buf, d0, ssem0, False)
            slot(2 * g + 1, r1buf, d1, ssem1, False)

        pltpu.make_async_copy(r0buf, acc.at[d0], ssem0).wait()
        build(nchunks - 1, r0buf, d0)
        pltpu.sync_copy(r0buf, acc.at[d0], add=True)
        pltpu.make_async_copy(r1buf, acc.at[d1], ssem1).wait()

        plsc.subcore_barrier()

        @pl.when(sid < ND)
        def _():
            pltpu.sync_copy(
                acc.at[pl.ds(sid * npsc, npsc)],
                out_hbm.at[cid, pl.ds(sid * npsc, npsc)],
            )

    return k


NR = 4
NSLOT = 8


@functools.lru_cache(maxsize=None)
def _edge_call(n, e, f):
    epw = e // NW
    nchunks = epw // CHUNK
    npsc = n // ND
    assert nchunks >= 2 * NSLOT
    body_end = ((nchunks - 2) // NSLOT) * NSLOT

    @functools.partial(
        pl.kernel,
        out_type=jax.ShapeDtypeStruct((NC, n, f), jnp.float32),
        mesh=_sc_mesh(),
        scratch_types=(
            [pltpu.VMEM((CHUNK, f), jnp.float32)] * NR
            + [pltpu.VMEM((CHUNK,), jnp.int32)] * NSLOT
            + [pltpu.VMEM((CHUNK,), jnp.int32)] * NSLOT
            + [pltpu.VMEM((CHUNK,), jnp.float32)] * NSLOT
            + [
                pltpu.VMEM_SHARED((n, f), jnp.float32),
                pltpu.SemaphoreType.DMA,
            ]
            + [pltpu.SemaphoreType.DMA] * NR
            + [pltpu.SemaphoreType.DMA] * NR
            + [pltpu.SemaphoreType.DMA] * NSLOT
        ),
    )
    def k(src_hbm, dst_hbm, ew_hbm, z_hbm, zeros_hbm, out_hbm, *rest):
        rows = list(rest[0:NR])
        o = NR
        srcb = list(rest[o:o + NSLOT]); o += NSLOT
        dstb = list(rest[o:o + NSLOT]); o += NSLOT
        ewb = list(rest[o:o + NSLOT]); o += NSLOT
        acc = rest[o]; o += 1
        zsem = rest[o]; o += 1
        gsem = list(rest[o:o + NR]); o += NR
        ssem = list(rest[o:o + NR]); o += NR
        stsem = list(rest[o:o + NSLOT])

        cid = lax.axis_index("c")
        sid = lax.axis_index("s")
        wid = cid * NS + sid
        base = pl.multiple_of(wid * epw, 8)

        def stissue(ci, c):
            pltpu.async_copy(src_hbm.at[pl.ds(base + ci * CHUNK, CHUNK)],
                             srcb[c], stsem[c])
            pltpu.async_copy(dst_hbm.at[pl.ds(base + ci * CHUNK, CHUNK)],
                             dstb[c], stsem[c])
            pltpu.async_copy(ew_hbm.at[pl.ds(base + ci * CHUNK, CHUNK)],
                             ewb[c], stsem[c])

        def stwait(c):
            pltpu.make_async_copy(src_hbm.at[pl.ds(base, CHUNK)],
                                  srcb[c], stsem[c]).wait()
            pltpu.make_async_copy(dst_hbm.at[pl.ds(base, CHUNK)],
                                  dstb[c], stsem[c]).wait()
            pltpu.make_async_copy(ew_hbm.at[pl.ds(base, CHUNK)],
                                  ewb[c], stsem[c]).wait()

        def gissue(b, c):
            pltpu.async_copy(z_hbm.at[srcb[c]], rows[b], gsem[b])

        def gwait(b, c):
            pltpu.make_async_copy(z_hbm.at[srcb[c]], rows[b], gsem[b]).wait()

        def scale(b, c):
            @pl.loop(0, CHUNK, step=LANES)
            def _(r0):
                ev = ewb[c][pl.ds(r0, LANES)]
                for j in range(LANES):
                    s = ev[j]
                    for kk in range(f // LANES):
                        sl = pl.ds(kk * LANES, LANES)
                        rows[b][r0 + j, sl] = rows[b][r0 + j, sl] * s

        def sissue(b, c):
            pltpu.async_copy(rows[b], acc.at[dstb[c]], ssem[b], add=True)

        def swait(b, c):
            pltpu.make_async_copy(rows[b], acc.at[dstb[c]], ssem[b]).wait()

        def emit(ci, k_, tail):
            b = k_ % NR
            gwait(b, k_)
            if not tail:
                stwait((k_ + 2) % NSLOT)
                gissue((k_ + 2) % NR, (k_ + 2) % NSLOT)

                @pl.when(ci + 6 < nchunks)
                def _():
                    stissue(ci + 6, (k_ + 6) % NSLOT)
            else:
                if ci + 2 < nchunks:
                    stwait((k_ + 2) % NSLOT)
                    gissue((k_ + 2) % NR, (k_ + 2) % NSLOT)
            scale(b, k_)
            if not tail:
                @pl.when(ci >= 1)
                def _():
                    swait((k_ + 3) % NR, (k_ + 7) % NSLOT)
            else:
                swait((k_ + 3) % NR, (k_ + 7) % NSLOT)
            sissue(b, k_)

        @pl.when(sid < ND)
        def _():
            pltpu.async_copy(zeros_hbm, acc.at[pl.ds(sid * npsc, npsc)], zsem)

        for cj in range(6):
            stissue(cj, cj)

        @pl.when(sid < ND)
        def _():
            pltpu.make_async_copy(
                zeros_hbm, acc.at[pl.ds(sid * npsc, npsc)], zsem
            ).wait()

        plsc.subcore_barrier()

        for cj in range(2):
            stwait(cj)
            gissue(cj, cj)

        @pl.loop(0, body_end // NSLOT)
        def _(g):
            for k_ in range(NSLOT):
                emit(g * NSLOT + k_, k_, False)

        for ci in range(body_end, nchunks):
            emit(ci, ci % NSLOT, True)

        swait((nchunks - 1) % NR, (nchunks - 1) % NSLOT)

        plsc.subcore_barrier()

        @pl.when(sid < ND)
        def _():
            pltpu.sync_copy(
                acc.at[pl.ds(sid * npsc, npsc)],
                out_hbm.at[cid, pl.ds(sid * npsc, npsc)],
            )

    return k


def _prep_body(x_ref, w0_ref, wih_ref, bih_ref, bhh_ref, deg_ref, z_ref, w_scr):
    i = pl.program_id(0)

    @pl.when(i == 0)
    def _():
        gates = (
            jnp.dot(w0_ref[...], wih_ref[...].T, preferred_element_type=jnp.float32)
            + bih_ref[...]
            + bhh_ref[...]
        )
        fi = gates[:, 0:128]
        fg = gates[:, 256:384]
        fo = gates[:, 384:512]
        c = jax.nn.sigmoid(fi) * jnp.tanh(fg)
        w_scr[...] = jax.nn.sigmoid(fo) * jnp.tanh(c)

    xw = jnp.dot(x_ref[...], w_scr[...], preferred_element_type=jnp.float32)
    deg = deg_ref[0, :, 0] + deg_ref[1, :, 0] + 1.0
    dinv = lax.rsqrt(deg)
    z_ref[...] = xw * dinv[:, None]


def _prep(x, w0, wih, bih, bhh, deg16):
    n, f = x.shape
    blk = 1000
    grid = n // blk
    return pl.pallas_call(
        _prep_body,
        grid=(grid,),
        in_specs=[
            pl.BlockSpec((blk, f), lambda i: (i, 0)),
            pl.BlockSpec((f, f), lambda i: (0, 0)),
            pl.BlockSpec((4 * f, f), lambda i: (0, 0)),
            pl.BlockSpec((1, 4 * f), lambda i: (0, 0)),
            pl.BlockSpec((1, 4 * f), lambda i: (0, 0)),
            pl.BlockSpec((NC, blk, LANES), lambda i: (0, i, 0)),
        ],
        out_specs=pl.BlockSpec((blk, f), lambda i: (i, 0)),
        out_shape=jax.ShapeDtypeStruct((n, f), jnp.float32),
        scratch_shapes=[pltpu.VMEM((f, f), jnp.float32)],
    )(x, w0, wih, bih.reshape(1, 4 * f), bhh.reshape(1, 4 * f), deg16)


def _final_body(p_ref, z_ref, deg_ref, linw_ref, linb_ref, out_ref):
    deg = deg_ref[0, :, 0] + deg_ref[1, :, 0] + 1.0
    dinv = lax.rsqrt(deg)
    h = jnp.maximum((p_ref[0] + p_ref[1] + z_ref[...]) * dinv[:, None], 0.0)
    logits = (
        jnp.dot(h, linw_ref[...].T, preferred_element_type=jnp.float32)
        + linb_ref[...]
    )
    m = jnp.max(logits, axis=1, keepdims=True)
    ex = jnp.exp(logits - m)
    out_ref[...] = ex / jnp.sum(ex, axis=1, keepdims=True)


def _final(p, z, deg16, linw, linb):
    n, f = z.shape
    ncls = linw.shape[0]
    blk = 1000
    grid = n // blk
    return pl.pallas_call(
        _final_body,
        grid=(grid,),
        in_specs=[
            pl.BlockSpec((NC, blk, f), lambda i: (0, i, 0)),
            pl.BlockSpec((blk, f), lambda i: (i, 0)),
            pl.BlockSpec((NC, blk, LANES), lambda i: (0, i, 0)),
            pl.BlockSpec((ncls, f), lambda i: (0, 0)),
            pl.BlockSpec((1, ncls), lambda i: (0, 0)),
        ],
        out_specs=pl.BlockSpec((blk, ncls), lambda i: (i, 0)),
        out_shape=jax.ShapeDtypeStruct((n, ncls), jnp.float32),
    )(p, z, deg16, linw, linb.reshape(1, ncls))


def kernel(x, edge_index, edge_weight, initial_weight, W_ih, W_hh, b_ih, b_hh,
           lin_W, lin_b):
    n, f = x.shape
    e = edge_weight.shape[0]
    src = edge_index[0]
    dst = edge_index[1]
    zeros_deg = jnp.zeros((n // ND, LANES), jnp.float32)
    zeros_main = jnp.zeros((n // ND, f), jnp.float32)

    deg16 = _deg_call(n, e)(dst, edge_weight, zeros_deg)
    z = _prep(x, initial_weight, W_ih, b_ih, b_hh, deg16)
    p = _edge_call(n, e, f)(src, dst, edge_weight, z, zeros_main)
    return _final(p, z, deg16, lin_W, lin_b)


# speedup vs baseline: 37.2112x; 1.0206x over previous
import functools

import jax
import jax.numpy as jnp
from jax import lax
from jax.experimental import pallas as pl
from jax.experimental.pallas import tpu as pltpu
from jax.experimental.pallas import tpu_sc as plsc

NC = 2
NS = 16
NW = NC * NS
LANES = 16
CHUNK = 80
ND = 10


def _sc_mesh():
    return plsc.VectorSubcoreMesh(
        core_axis_name="c", subcore_axis_name="s", num_cores=NC, num_subcores=NS
    )


@functools.lru_cache(maxsize=None)
def _deg_call(n, e):
    epw = e // NW
    nchunks = epw // CHUNK
    assert nchunks % 2 == 1
    npairs = nchunks // 2
    npsc = n // ND

    @functools.partial(
        pl.kernel,
        out_type=jax.ShapeDtypeStruct((NC, n, LANES), jnp.float32),
        mesh=_sc_mesh(),
        scratch_types=[
            pltpu.VMEM((epw,), jnp.int32),
            pltpu.VMEM((epw,), jnp.float32),
            pltpu.VMEM((CHUNK, LANES), jnp.float32),
            pltpu.VMEM((CHUNK, LANES), jnp.float32),
            pltpu.VMEM((CHUNK,), jnp.int32),
            pltpu.VMEM((CHUNK,), jnp.int32),
            pltpu.VMEM_SHARED((n, LANES), jnp.float32),
            pltpu.SemaphoreType.DMA,
            pltpu.SemaphoreType.DMA,
            pltpu.SemaphoreType.DMA,
        ],
    )
    def k(dst_hbm, ew_hbm, zeros_hbm, out_hbm,
          dsts, ews, r0buf, r1buf, d0, d1, acc, stsem, ssem0, ssem1):
        cid = lax.axis_index("c")
        sid = lax.axis_index("s")
        wid = cid * NS + sid
        base = pl.multiple_of(wid * epw, 8)

        pltpu.async_copy(dst_hbm.at[pl.ds(base, epw)], dsts, stsem)
        pltpu.async_copy(ew_hbm.at[pl.ds(base, epw)], ews, stsem)

        @pl.when(sid < ND)
        def _():
            pltpu.async_copy(zeros_hbm, acc.at[pl.ds(sid * npsc, npsc)], stsem)

        pltpu.make_async_copy(dst_hbm.at[pl.ds(base, epw)], dsts, stsem).wait()
        pltpu.make_async_copy(ew_hbm.at[pl.ds(base, epw)], ews, stsem).wait()

        @pl.when(sid < ND)
        def _():
            pltpu.make_async_copy(
                zeros_hbm, acc.at[pl.ds(sid * npsc, npsc)], stsem
            ).wait()

        plsc.subcore_barrier()

        def build(ci, rbuf, dcur):
            @pl.loop(0, CHUNK, step=LANES)
            def _(r0):
                ev = ews[pl.ds(ci * CHUNK + r0, LANES)]
                for j in range(LANES):
                    rbuf[r0 + j, :] = jnp.broadcast_to(ev[j], (LANES,))

            for t in range(CHUNK // LANES):
                sl = pl.ds(t * LANES, LANES)
                dcur[sl] = dsts[pl.ds(ci * CHUNK + t * LANES, LANES)]

        def slot(ci, rbuf, dcur, ssem, first):
            if not first:
                pltpu.make_async_copy(rbuf, acc.at[dcur], ssem).wait()
            build(ci, rbuf, dcur)
            pltpu.async_copy(rbuf, acc.at[dcur], ssem, add=True)

        slot(0, r0buf, d0, ssem0, True)
        slot(1, r1buf, d1, ssem1, True)

        @pl.loop(1, npairs)
        def _(g):
            slot(2 * g, r0buf, d0, ssem0, False)
            slot(2 * g + 1, r1buf, d1, ssem1, False)

        pltpu.make_async_copy(r0buf, acc.at[d0], ssem0).wait()
        build(nchunks - 1, r0buf, d0)
        pltpu.sync_copy(r0buf, acc.at[d0], add=True)
        pltpu.make_async_copy(r1buf, acc.at[d1], ssem1).wait()

        plsc.subcore_barrier()

        @pl.when(sid < ND)
        def _():
            pltpu.sync_copy(
                acc.at[pl.ds(sid * npsc, npsc)],
                out_hbm.at[cid, pl.ds(sid * npsc, npsc)],
            )

    return k


NR = 4
NSLOT = 8


@functools.lru_cache(maxsize=None)
def _edge_call(n, e, f):
    epw = e // NW
    nchunks = epw // CHUNK
    npsc = n // ND
    assert nchunks >= 2 * NSLOT
    body_end = ((nchunks - 2) // NSLOT) * NSLOT

    @functools.partial(
        pl.kernel,
        out_type=jax.ShapeDtypeStruct((NC, n, f), jnp.float32),
        mesh=_sc_mesh(),
        scratch_types=(
            [pltpu.VMEM((CHUNK, f), jnp.float32)] * NR
            + [pltpu.VMEM((CHUNK,), jnp.int32)] * NSLOT
            + [pltpu.VMEM((CHUNK,), jnp.int32)] * NSLOT
            + [pltpu.VMEM((CHUNK,), jnp.float32)] * NSLOT
            + [
                pltpu.VMEM_SHARED((n, f), jnp.float32),
                pltpu.SemaphoreType.DMA,
            ]
            + [pltpu.SemaphoreType.DMA] * NR
            + [pltpu.SemaphoreType.DMA] * NR
            + [pltpu.SemaphoreType.DMA] * NSLOT
        ),
    )
    def k(src_hbm, dst_hbm, ew_hbm, z_hbm, zeros_hbm, out_hbm, *rest):
        rows = list(rest[0:NR])
        o = NR
        srcb = list(rest[o:o + NSLOT]); o += NSLOT
        dstb = list(rest[o:o + NSLOT]); o += NSLOT
        ewb = list(rest[o:o + NSLOT]); o += NSLOT
        acc = rest[o]; o += 1
        zsem = rest[o]; o += 1
        gsem = list(rest[o:o + NR]); o += NR
        ssem = list(rest[o:o + NR]); o += NR
        stsem = list(rest[o:o + NSLOT])

        cid = lax.axis_index("c")
        sid = lax.axis_index("s")
        wid = cid * NS + sid
        base = pl.multiple_of(wid * epw, 8)

        def stissue(ci, c):
            pltpu.async_copy(src_hbm.at[pl.ds(base + ci * CHUNK, CHUNK)],
                             srcb[c], stsem[c])
            pltpu.async_copy(dst_hbm.at[pl.ds(base + ci * CHUNK, CHUNK)],
                             dstb[c], stsem[c])
            pltpu.async_copy(ew_hbm.at[pl.ds(base + ci * CHUNK, CHUNK)],
                             ewb[c], stsem[c])

        def stwait(c):
            pltpu.make_async_copy(src_hbm.at[pl.ds(base, CHUNK)],
                                  srcb[c], stsem[c]).wait()
            pltpu.make_async_copy(dst_hbm.at[pl.ds(base, CHUNK)],
                                  dstb[c], stsem[c]).wait()
            pltpu.make_async_copy(ew_hbm.at[pl.ds(base, CHUNK)],
                                  ewb[c], stsem[c]).wait()

        def gissue(b, c):
            pltpu.async_copy(z_hbm.at[srcb[c]], rows[b], gsem[b])

        def gwait(b, c):
            pltpu.make_async_copy(z_hbm.at[srcb[c]], rows[b], gsem[b]).wait()

        def scale(b, c):
            @pl.loop(0, CHUNK, step=LANES)
            def _(r0):
                ev = ewb[c][pl.ds(r0, LANES)]
                for j in range(LANES):
                    s = ev[j]
                    for kk in range(f // LANES):
                        sl = pl.ds(kk * LANES, LANES)
                        rows[b][r0 + j, sl] = rows[b][r0 + j, sl] * s

        def sissue(b, c):
            pltpu.async_copy(rows[b], acc.at[dstb[c]], ssem[b], add=True)

        def swait(b, c):
            pltpu.make_async_copy(rows[b], acc.at[dstb[c]], ssem[b]).wait()

        def emit(ci, k_, tail):
            b = k_ % NR
            gwait(b, k_)
            if not tail:
                stwait((k_ + 2) % NSLOT)
                gissue((k_ + 2) % NR, (k_ + 2) % NSLOT)

                @pl.when(ci + 6 < nchunks)
                def _():
                    stissue(ci + 6, (k_ + 6) % NSLOT)
            else:
                if ci + 2 < nchunks:
                    stwait((k_ + 2) % NSLOT)
                    gissue((k_ + 2) % NR, (k_ + 2) % NSLOT)
            scale(b, k_)
            if not tail:
                @pl.when(ci >= 1)
                def _():
                    swait((k_ + 3) % NR, (k_ + 7) % NSLOT)
            else:
                swait((k_ + 3) % NR, (k_ + 7) % NSLOT)
            sissue(b, k_)

        @pl.when(sid < ND)
        def _():
            pltpu.async_copy(zeros_hbm, acc.at[pl.ds(sid * npsc, npsc)], zsem)

        for cj in range(6):
            stissue(cj, cj)

        @pl.when(sid < ND)
        def _():
            pltpu.make_async_copy(
                zeros_hbm, acc.at[pl.ds(sid * npsc, npsc)], zsem
            ).wait()

        plsc.subcore_barrier()

        for cj in range(2):
            stwait(cj)
            gissue(cj, cj)

        @pl.loop(0, body_end // NSLOT)
        def _(g):
            for k_ in range(NSLOT):
                emit(g * NSLOT + k_, k_, False)

        for ci in range(body_end, nchunks):
            emit(ci, ci % NSLOT, True)

        swait((nchunks - 1) % NR, (nchunks - 1) % NSLOT)

        plsc.subcore_barrier()

        @pl.when(sid < ND)
        def _():
            pltpu.sync_copy(
                acc.at[pl.ds(sid * npsc, npsc)],
                out_hbm.at[cid, pl.ds(sid * npsc, npsc)],
            )

    return k


def _evolve_xw_body(x_ref, w0_ref, wih_ref, bih_ref, bhh_ref, xw_ref, w_scr):
    i = pl.program_id(0)

    @pl.when(i == 0)
    def _():
        gates = (
            jnp.dot(w0_ref[...], wih_ref[...].T, preferred_element_type=jnp.float32)
            + bih_ref[...]
            + bhh_ref[...]
        )
        fi = gates[:, 0:128]
        fg = gates[:, 256:384]
        fo = gates[:, 384:512]
        c = jax.nn.sigmoid(fi) * jnp.tanh(fg)
        w_scr[...] = jax.nn.sigmoid(fo) * jnp.tanh(c)

    xw_ref[...] = jnp.dot(x_ref[...], w_scr[...], preferred_element_type=jnp.float32)


def _evolve_xw(x, w0, wih, bih, bhh):
    n, f = x.shape
    blk = 2000
    grid = n // blk
    return pl.pallas_call(
        _evolve_xw_body,
        grid=(grid,),
        in_specs=[
            pl.BlockSpec((blk, f), lambda i: (i, 0)),
            pl.BlockSpec((f, f), lambda i: (0, 0)),
            pl.BlockSpec((4 * f, f), lambda i: (0, 0)),
            pl.BlockSpec((1, 4 * f), lambda i: (0, 0)),
            pl.BlockSpec((1, 4 * f), lambda i: (0, 0)),
        ],
        out_specs=pl.BlockSpec((blk, f), lambda i: (i, 0)),
        out_shape=jax.ShapeDtypeStruct((n, f), jnp.float32),
        scratch_shapes=[pltpu.VMEM((f, f), jnp.float32)],
    )(x, w0, wih, bih.reshape(1, 4 * f), bhh.reshape(1, 4 * f))


def _zscale_body(xw_ref, deg_ref, z_ref):
    deg = deg_ref[0, :, 0] + deg_ref[1, :, 0] + 1.0
    z_ref[...] = xw_ref[...] * lax.rsqrt(deg)[:, None]


def _zscale(xw, deg16):
    n, f = xw.shape
    blk = 5000
    grid = n // blk
    return pl.pallas_call(
        _zscale_body,
        grid=(grid,),
        in_specs=[
            pl.BlockSpec((blk, f), lambda i: (i, 0)),
            pl.BlockSpec((NC, blk, LANES), lambda i: (0, i, 0)),
        ],
        out_specs=pl.BlockSpec((blk, f), lambda i: (i, 0)),
        out_shape=jax.ShapeDtypeStruct((n, f), jnp.float32),
    )(xw, deg16)


def _final_body(p_ref, z_ref, deg_ref, linw_ref, linb_ref, out_ref):
    deg = deg_ref[0, :, 0] + deg_ref[1, :, 0] + 1.0
    dinv = lax.rsqrt(deg)
    h = jnp.maximum((p_ref[0] + p_ref[1] + z_ref[...]) * dinv[:, None], 0.0)
    logits = (
        jnp.dot(h, linw_ref[...].T, preferred_element_type=jnp.float32)
        + linb_ref[...]
    )
    m = jnp.max(logits, axis=1, keepdims=True)
    ex = jnp.exp(logits - m)
    out_ref[...] = ex / jnp.sum(ex, axis=1, keepdims=True)


def _final(p, z, deg16, linw, linb):
    n, f = z.shape
    ncls = linw.shape[0]
    blk = 2000
    grid = n // blk
    return pl.pallas_call(
        _final_body,
        grid=(grid,),
        in_specs=[
            pl.BlockSpec((NC, blk, f), lambda i: (0, i, 0)),
            pl.BlockSpec((blk, f), lambda i: (i, 0)),
            pl.BlockSpec((NC, blk, LANES), lambda i: (0, i, 0)),
            pl.BlockSpec((ncls, f), lambda i: (0, 0)),
            pl.BlockSpec((1, ncls), lambda i: (0, 0)),
        ],
        out_specs=pl.BlockSpec((blk, ncls), lambda i: (i, 0)),
        out_shape=jax.ShapeDtypeStruct((n, ncls), jnp.float32),
    )(p, z, deg16, linw, linb.reshape(1, ncls))


def kernel(x, edge_index, edge_weight, initial_weight, W_ih, W_hh, b_ih, b_hh,
           lin_W, lin_b):
    n, f = x.shape
    e = edge_weight.shape[0]
    src = edge_index[0]
    dst = edge_index[1]
    zeros_deg = jnp.zeros((n // ND, LANES), jnp.float32)
    zeros_main = jnp.zeros((n // ND, f), jnp.float32)

    deg16 = _deg_call(n, e)(dst, edge_weight, zeros_deg)
    xw = _evolve_xw(x, initial_weight, W_ih, b_ih, b_hh)
    z = _zscale(xw, deg16)
    p = _edge_call(n, e, f)(src, dst, edge_weight, z, zeros_main)
    return _final(p, z, deg16, lin_W, lin_b)
